# Initial kernel scaffold; baseline (speedup 1.0000x reference)
import jax
import jax.numpy as jnp
from jax.experimental import pallas as pl


def kernel(input_ids, embedding_weight):
    raise NotImplementedError("write your pallas kernel here")


# speedup vs baseline: 1.4752x; 1.4752x over previous
import jax
import jax.numpy as jnp
from jax import lax
from jax.experimental import pallas as pl
from jax.experimental.pallas import tpu as pltpu
from jax.experimental.pallas import tpu_sc as plsc

EMBED_DIM = 128
CHUNK = 128
N_CHUNK = 4


def _emb_body(idx_hbm, table_hbm, out_hbm, idx_v, rows_v, sem):
    num_cores = plsc.get_sparse_core_info().num_cores
    wid = lax.axis_index("s") * num_cores + lax.axis_index("c")
    pltpu.sync_copy(idx_hbm.at[wid], idx_v)
    copies = []
    for j in range(N_CHUNK):
        copies.append(
            pltpu.async_copy(
                table_hbm.at[idx_v.at[j]],
                rows_v.at[pl.ds(j * CHUNK, CHUNK)],
                sem,
            )
        )
    for c in copies:
        c.wait()
    pltpu.sync_copy(rows_v, out_hbm.at[pl.ds(wid * N_CHUNK * CHUNK, N_CHUNK * CHUNK)])


def kernel(input_ids, embedding_weight):
    batch, seq_len = input_ids.shape
    n = batch * seq_len
    info = plsc.get_sparse_core_info()
    nw = info.num_cores * info.num_subcores
    assert n == nw * N_CHUNK * CHUNK
    idx = input_ids.reshape(nw, N_CHUNK, CHUNK).astype(jnp.int32)
    mesh = plsc.VectorSubcoreMesh(core_axis_name="c", subcore_axis_name="s")
    out = pl.kernel(
        _emb_body,
        mesh=mesh,
        out_type=jax.ShapeDtypeStruct((n, EMBED_DIM), jnp.float32),
        scratch_types=[
            pltpu.VMEM((N_CHUNK, CHUNK), jnp.int32),
            pltpu.VMEM((N_CHUNK * CHUNK, EMBED_DIM), jnp.float32),
            pltpu.SemaphoreType.DMA,
        ],
    )(idx, embedding_weight)
    return out.reshape(batch, seq_len, EMBED_DIM)


# speedup vs baseline: 1.4778x; 1.0018x over previous
import jax
import jax.numpy as jnp
from jax import lax
from jax.experimental import pallas as pl
from jax.experimental.pallas import tpu as pltpu
from jax.experimental.pallas import tpu_sc as plsc

EMBED_DIM = 128
CHUNK = 128
N_CHUNK = 4


def _emb_body(idx_hbm, table_hbm, out_hbm, idx_v, rows_v, wsem, *gsems):
    num_cores = plsc.get_sparse_core_info().num_cores
    wid = lax.axis_index("s") * num_cores + lax.axis_index("c")
    base = wid * N_CHUNK * CHUNK
    pltpu.sync_copy(idx_hbm.at[wid], idx_v)
    gathers = [
        pltpu.async_copy(table_hbm.at[idx_v.at[j]], rows_v.at[j], gsems[j])
        for j in range(N_CHUNK)
    ]
    writes = []
    for j in range(N_CHUNK):
        gathers[j].wait()
        writes.append(
            pltpu.async_copy(
                rows_v.at[j], out_hbm.at[pl.ds(base + j * CHUNK, CHUNK)], wsem
            )
        )
    for c in writes:
        c.wait()


def kernel(input_ids, embedding_weight):
    batch, seq_len = input_ids.shape
    n = batch * seq_len
    info = plsc.get_sparse_core_info()
    nw = info.num_cores * info.num_subcores
    assert n == nw * N_CHUNK * CHUNK
    idx = input_ids.reshape(nw, N_CHUNK, CHUNK).astype(jnp.int32)
    mesh = plsc.VectorSubcoreMesh(core_axis_name="c", subcore_axis_name="s")
    out = pl.kernel(
        _emb_body,
        mesh=mesh,
        out_type=jax.ShapeDtypeStruct((n, EMBED_DIM), jnp.float32),
        scratch_types=[
            pltpu.VMEM((N_CHUNK, CHUNK), jnp.int32),
            pltpu.VMEM((N_CHUNK, CHUNK, EMBED_DIM), jnp.float32),
            pltpu.SemaphoreType.DMA,
        ]
        + [pltpu.SemaphoreType.DMA] * N_CHUNK,
    )(idx, embedding_weight)
    return out.reshape(batch, seq_len, EMBED_DIM)


# speedup vs baseline: 1.4857x; 1.0054x over previous
import jax
import jax.numpy as jnp
from jax import lax
from jax.experimental import pallas as pl
from jax.experimental.pallas import tpu as pltpu
from jax.experimental.pallas import tpu_sc as plsc

EMBED_DIM = 128
CHUNK = 512
N_CHUNK = 1


def _emb_body(idx_hbm, table_hbm, out_hbm, idx_v, rows_v, wsem, *gsems):
    num_cores = plsc.get_sparse_core_info().num_cores
    wid = lax.axis_index("s") * num_cores + lax.axis_index("c")
    base = wid * N_CHUNK * CHUNK
    pltpu.sync_copy(idx_hbm.at[wid], idx_v)
    gathers = [
        pltpu.async_copy(table_hbm.at[idx_v.at[j]], rows_v.at[j], gsems[j])
        for j in range(N_CHUNK)
    ]
    writes = []
    for j in range(N_CHUNK):
        gathers[j].wait()
        writes.append(
            pltpu.async_copy(
                rows_v.at[j], out_hbm.at[pl.ds(base + j * CHUNK, CHUNK)], wsem
            )
        )
    for c in writes:
        c.wait()


def kernel(input_ids, embedding_weight):
    batch, seq_len = input_ids.shape
    n = batch * seq_len
    info = plsc.get_sparse_core_info()
    nw = info.num_cores * info.num_subcores
    assert n == nw * N_CHUNK * CHUNK
    idx = input_ids.reshape(nw, N_CHUNK, CHUNK).astype(jnp.int32)
    mesh = plsc.VectorSubcoreMesh(core_axis_name="c", subcore_axis_name="s")
    out = pl.kernel(
        _emb_body,
        mesh=mesh,
        out_type=jax.ShapeDtypeStruct((n, EMBED_DIM), jnp.float32),
        scratch_types=[
            pltpu.VMEM((N_CHUNK, CHUNK), jnp.int32),
            pltpu.VMEM((N_CHUNK, CHUNK, EMBED_DIM), jnp.float32),
            pltpu.SemaphoreType.DMA,
        ]
        + [pltpu.SemaphoreType.DMA] * N_CHUNK,
    )(idx, embedding_weight)
    return out.reshape(batch, seq_len, EMBED_DIM)


# speedup vs baseline: 1.4946x; 1.0060x over previous
import jax
import jax.numpy as jnp
from jax import lax
from jax.experimental import pallas as pl
from jax.experimental.pallas import tpu as pltpu
from jax.experimental.pallas import tpu_sc as plsc

EMBED_DIM = 128


def _emb_body(idx_hbm, table_hbm, out_hbm, idx_v, rows_v, sem):
    info = plsc.get_sparse_core_info()
    nw = info.num_cores * info.num_subcores
    seq_len = idx_hbm.shape[1]
    per = (idx_hbm.shape[0] * seq_len) // nw
    tiles_per_row = seq_len // per
    wid = lax.axis_index("s") * info.num_cores + lax.axis_index("c")
    b = wid // tiles_per_row
    c0 = (wid % tiles_per_row) * per
    pltpu.sync_copy(idx_hbm.at[b, pl.ds(c0, per)], idx_v)
    pltpu.async_copy(table_hbm.at[idx_v], rows_v, sem).wait()
    pltpu.sync_copy(rows_v, out_hbm.at[b, pl.ds(c0, per)])


def kernel(input_ids, embedding_weight):
    batch, seq_len = input_ids.shape
    mesh = plsc.VectorSubcoreMesh(core_axis_name="c", subcore_axis_name="s")
    info = plsc.get_sparse_core_info()
    per = (batch * seq_len) // (info.num_cores * info.num_subcores)
    return pl.kernel(
        _emb_body,
        mesh=mesh,
        out_type=jax.ShapeDtypeStruct((batch, seq_len, EMBED_DIM), jnp.float32),
        scratch_types=[
            pltpu.VMEM((per,), jnp.int32),
            pltpu.VMEM((per, EMBED_DIM), jnp.float32),
            pltpu.SemaphoreType.DMA,
        ],
    )(input_ids.astype(jnp.int32), embedding_weight)
